# baseline (speedup 1.0000x reference)
import functools

import jax
import jax.numpy as jnp
from jax import lax
from jax.experimental import pallas as pl
from jax.experimental.pallas import tpu as pltpu
from jax.experimental.pallas import tpu_sc as plsc

NUM_VOCAB = 100000
EMBED_DIM = 64
N, S, A = 4096, 50, 3
B = N * S
L = 16

NC = 2
NS = 16
NW = NC * NS
B_PER_W = B // NW
SUB = 128
CHUNK = 640
N_SUB = CHUNK // SUB
N_CHUNK = B_PER_W // CHUNK


def _sc_gather(value_flat, table):
    mesh = plsc.VectorSubcoreMesh(core_axis_name="c", subcore_axis_name="s")

    @functools.partial(
        pl.kernel,
        mesh=mesh,
        out_type=jax.ShapeDtypeStruct((B, EMBED_DIM), jnp.float32),
        compiler_params=pltpu.CompilerParams(use_tc_tiling_on_sc=False),
        scratch_types=[
            pltpu.VMEM((B_PER_W,), jnp.int32),
            pltpu.VMEM((CHUNK, EMBED_DIM), jnp.float32),
            pltpu.VMEM((CHUNK, EMBED_DIM), jnp.float32),
            pltpu.SemaphoreType.DMA,
            pltpu.SemaphoreType.DMA,
            pltpu.SemaphoreType.DMA,
            pltpu.SemaphoreType.DMA,
        ],
    )
    def gather_kernel(value_hbm, table_hbm, out_hbm,
                      idx_v, rows_a, rows_b, gsem_a, gsem_b, osem_a, osem_b):
        wid = lax.axis_index("s") * NC + lax.axis_index("c")
        base = wid * B_PER_W
        pltpu.sync_copy(value_hbm.at[pl.ds(base, B_PER_W)], idx_v)
        rows = (rows_a, rows_b)
        gsem = (gsem_a, gsem_b)
        osem = (osem_a, osem_b)

        def fire_gathers(j, buf, sem):
            return [
                pltpu.async_copy(
                    table_hbm.at[idx_v.at[pl.ds(j * CHUNK + k * SUB, SUB)]],
                    buf.at[pl.ds(k * SUB, SUB)],
                    sem,
                )
                for k in range(N_SUB)
            ]

        pending_g = fire_gathers(0, rows[0], gsem[0])
        pending_o = [None, None]
        for j in range(N_CHUNK):
            cur = j % 2
            nxt = (j + 1) % 2
            if j + 1 < N_CHUNK:
                if pending_o[nxt] is not None:
                    pending_o[nxt].wait()
                    pending_o[nxt] = None
                next_g = fire_gathers(j + 1, rows[nxt], gsem[nxt])
            for c in pending_g:
                c.wait()
            pending_o[cur] = pltpu.async_copy(
                rows[cur], out_hbm.at[pl.ds(base + j * CHUNK, CHUNK)], osem[cur]
            )
            if j + 1 < N_CHUNK:
                pending_g = next_g
        for o in pending_o:
            if o is not None:
                o.wait()

    return gather_kernel(value_flat, table)


H = N // 2


def _tc_add_transpose(g2, p2t, W2, eye):
    RPS = N * EMBED_DIM // 128

    def add_t_kernel(g_ref, p_ref, w_ref, e_ref, o_ref):
        x = g_ref[...]
        lin = lax.dot_general(
            p_ref[0], w_ref[...], (((0,), (0,)), ((), ())),
            preferred_element_type=jnp.float32,
            precision=jax.lax.Precision.HIGHEST)
        y = x + lin
        yT = lax.dot_general(
            e_ref[...], y, (((1,), (1,)), ((), ())),
            preferred_element_type=jnp.float32,
            precision=jax.lax.Precision.HIGHEST)
        o_ref[0, :, 0:H] = yT[0:EMBED_DIM, :]
        o_ref[0, :, H:N] = yT[EMBED_DIM:128, :]

    return pl.pallas_call(
        add_t_kernel,
        grid=(S,),
        in_specs=[
            pl.BlockSpec((RPS, 128), lambda i: (i, 0)),
            pl.BlockSpec((1, 8, H), lambda i: (i, 0, 0)),
            pl.BlockSpec((8, 128), lambda i: (0, 0)),
            pl.BlockSpec((128, 128), lambda i: (0, 0)),
        ],
        out_specs=pl.BlockSpec((1, EMBED_DIM, N), lambda i: (i, 0, 0)),
        out_shape=jax.ShapeDtypeStruct((S, EMBED_DIM, N), jnp.float32),
    )(g2, p2t, W2, eye)


def kernel(value, depth, position, table, W, b):
    del depth
    value_t = value.T
    value_flat = jnp.stack(
        [value_t[:, :H], value_t[:, H:]], axis=2).reshape(B)
    gathered = _sc_gather(value_flat, table)
    g2 = gathered.reshape(B * EMBED_DIM // 128, 128)
    pos_pl = position.transpose(2, 1, 0)
    ones = jnp.ones((1, S, H), jnp.float32)
    p2t = jnp.concatenate(
        [pos_pl[:, :, :H], ones, pos_pl[:, :, H:], ones],
        axis=0).transpose(1, 0, 2)
    z = jnp.zeros_like(W)
    zb = jnp.zeros_like(b)
    W2 = jnp.concatenate([
        jnp.concatenate([W, z], axis=1),
        jnp.concatenate([b.reshape(1, -1), zb.reshape(1, -1)], axis=1),
        jnp.concatenate([z, W], axis=1),
        jnp.concatenate([zb.reshape(1, -1), b.reshape(1, -1)], axis=1),
    ], axis=0)
    eye = jnp.eye(128, dtype=jnp.float32)
    out_t = _tc_add_transpose(g2, p2t, W2, eye)
    return jnp.transpose(out_t, (2, 0, 1))


# speedup vs baseline: 1.3901x; 1.3901x over previous
import functools

import jax
import jax.numpy as jnp
from jax import lax
from jax.experimental import pallas as pl
from jax.experimental.pallas import tpu as pltpu
from jax.experimental.pallas import tpu_sc as plsc

NUM_VOCAB = 100000
EMBED_DIM = 64
N, S, A = 4096, 50, 3
B = N * S
L = 16

NC = 2
NS = 16
NW = NC * NS
B_PER_W = B // NW
SUB = 128
CHUNK = 640
N_SUB = CHUNK // SUB
N_CHUNK = B_PER_W // CHUNK


def _sc_gather(value_flat, table):
    mesh = plsc.VectorSubcoreMesh(core_axis_name="c", subcore_axis_name="s")

    @functools.partial(
        pl.kernel,
        mesh=mesh,
        out_type=jax.ShapeDtypeStruct((B, EMBED_DIM), jnp.float32),
        compiler_params=pltpu.CompilerParams(use_tc_tiling_on_sc=False),
        scratch_types=[
            pltpu.VMEM((B_PER_W,), jnp.int32),
            pltpu.VMEM((CHUNK, EMBED_DIM), jnp.float32),
            pltpu.VMEM((CHUNK, EMBED_DIM), jnp.float32),
            pltpu.SemaphoreType.DMA,
            pltpu.SemaphoreType.DMA,
            pltpu.SemaphoreType.DMA,
            pltpu.SemaphoreType.DMA,
        ],
    )
    def gather_kernel(value_hbm, table_hbm, out_hbm,
                      idx_v, rows_a, rows_b, gsem_a, gsem_b, osem_a, osem_b):
        wid = lax.axis_index("s") * NC + lax.axis_index("c")
        base = wid * B_PER_W
        pltpu.sync_copy(value_hbm.at[pl.ds(base, B_PER_W)], idx_v)
        rows = (rows_a, rows_b)
        gsem = (gsem_a, gsem_b)
        osem = (osem_a, osem_b)

        def fire_gathers(j, buf, sem):
            return [
                pltpu.async_copy(
                    table_hbm.at[idx_v.at[pl.ds(j * CHUNK + k * SUB, SUB)]],
                    buf.at[pl.ds(k * SUB, SUB)],
                    sem,
                )
                for k in range(N_SUB)
            ]

        pending_g = fire_gathers(0, rows[0], gsem[0])
        pending_o = [None, None]
        for j in range(N_CHUNK):
            cur = j % 2
            nxt = (j + 1) % 2
            if j + 1 < N_CHUNK:
                if pending_o[nxt] is not None:
                    pending_o[nxt].wait()
                    pending_o[nxt] = None
                next_g = fire_gathers(j + 1, rows[nxt], gsem[nxt])
            for c in pending_g:
                c.wait()
            pending_o[cur] = pltpu.async_copy(
                rows[cur], out_hbm.at[pl.ds(base + j * CHUNK, CHUNK)], osem[cur]
            )
            if j + 1 < N_CHUNK:
                pending_g = next_g
        for o in pending_o:
            if o is not None:
                o.wait()

    return gather_kernel(value_flat, table)


H = N // 2


def _tc_add_transpose(g2, p2t, W2, eye):
    RPS = N * EMBED_DIM // 128

    def add_t_kernel(g_ref, p_ref, w_ref, e_ref, o_ref):
        x = g_ref[...]
        lin = lax.dot_general(
            p_ref[0], w_ref[...], (((0,), (0,)), ((), ())),
            preferred_element_type=jnp.float32,
            precision=jax.lax.Precision.HIGHEST)
        del e_ref
        yT = (x + lin).T
        o_ref[0, :, 0:H] = yT[0:EMBED_DIM, :]
        o_ref[0, :, H:N] = yT[EMBED_DIM:128, :]

    return pl.pallas_call(
        add_t_kernel,
        grid=(S,),
        in_specs=[
            pl.BlockSpec((RPS, 128), lambda i: (i, 0)),
            pl.BlockSpec((1, 8, H), lambda i: (i, 0, 0)),
            pl.BlockSpec((8, 128), lambda i: (0, 0)),
            pl.BlockSpec((128, 128), lambda i: (0, 0)),
        ],
        out_specs=pl.BlockSpec((1, EMBED_DIM, N), lambda i: (i, 0, 0)),
        out_shape=jax.ShapeDtypeStruct((S, EMBED_DIM, N), jnp.float32),
    )(g2, p2t, W2, eye)


def kernel(value, depth, position, table, W, b):
    del depth
    value_t = value.T
    value_flat = jnp.stack(
        [value_t[:, :H], value_t[:, H:]], axis=2).reshape(B)
    gathered = _sc_gather(value_flat, table)
    g2 = gathered.reshape(B * EMBED_DIM // 128, 128)
    pos_pl = position.transpose(2, 1, 0)
    ones = jnp.ones((1, S, H), jnp.float32)
    p2t = jnp.concatenate(
        [pos_pl[:, :, :H], ones, pos_pl[:, :, H:], ones],
        axis=0).transpose(1, 0, 2)
    z = jnp.zeros_like(W)
    zb = jnp.zeros_like(b)
    W2 = jnp.concatenate([
        jnp.concatenate([W, z], axis=1),
        jnp.concatenate([b.reshape(1, -1), zb.reshape(1, -1)], axis=1),
        jnp.concatenate([z, W], axis=1),
        jnp.concatenate([zb.reshape(1, -1), b.reshape(1, -1)], axis=1),
    ], axis=0)
    eye = jnp.eye(128, dtype=jnp.float32)
    out_t = _tc_add_transpose(g2, p2t, W2, eye)
    return jnp.transpose(out_t, (2, 0, 1))
